# Initial kernel scaffold; baseline (speedup 1.0000x reference)
import jax
import jax.numpy as jnp
from jax.experimental import pallas as pl


def kernel(x, edge_index, W1, b1, W2, b2):
    raise NotImplementedError("write your pallas kernel here")


# speedup vs baseline: 4.1740x; 4.1740x over previous
import functools

import jax
import jax.numpy as jnp
from jax import lax
from jax.experimental import pallas as pl
from jax.experimental.pallas import tpu as pltpu
from jax.experimental.pallas import tpu_sc as plsc

N_NODES = 10000
N_EDGES = 320000
D = 128

NC = 2
NS = 16
NW = NC * NS

CHUNK = 128
NCH = 80
EDGES_PER_W = NCH * CHUNK
E_PAD = EDGES_PER_W * NW
DUMMY = N_NODES
ACC_ROWS = 10112
RPT = ACC_ROWS // NS

_sc_mesh = plsc.VectorSubcoreMesh(core_axis_name="c", subcore_axis_name="s")


@functools.partial(
    pl.kernel,
    mesh=_sc_mesh,
    out_type=[
        jax.ShapeDtypeStruct((ACC_ROWS, D), jnp.float32),
        jax.ShapeDtypeStruct((ACC_ROWS, D), jnp.float32),
    ],
    scratch_types=[
        pltpu.VMEM((NCH, CHUNK), jnp.int32),
        pltpu.VMEM((NCH, CHUNK), jnp.int32),
        pltpu.VMEM((CHUNK, D), jnp.float32),
        pltpu.VMEM_SHARED((ACC_ROWS, D), jnp.float32),
        pltpu.SemaphoreType.DMA,
    ],
)
def _sc_agg(h_hbm, src_hbm, dst_hbm, zeros_hbm, p0_hbm, p1_hbm,
            src_v, dst_v, rows_v, acc, sem):
    c = lax.axis_index("c")
    s = lax.axis_index("s")

    pltpu.sync_copy(src_hbm.at[c, s], src_v)
    pltpu.sync_copy(dst_hbm.at[c, s], dst_v)
    pltpu.sync_copy(zeros_hbm.at[pl.ds(s * RPT, RPT)],
                    acc.at[pl.ds(s * RPT, RPT)])
    plsc.subcore_barrier()

    def body(j, carry):
        pltpu.async_copy(h_hbm.at[src_v.at[j]], rows_v, sem).wait()
        pltpu.sync_copy(rows_v, acc.at[dst_v.at[j]], add=True)
        return carry

    lax.fori_loop(0, NCH, body, 0)
    plsc.subcore_barrier()

    @pl.when(c == 0)
    def _():
        pltpu.sync_copy(acc.at[pl.ds(s * RPT, RPT)],
                        p0_hbm.at[pl.ds(s * RPT, RPT)])

    @pl.when(c == 1)
    def _():
        pltpu.sync_copy(acc.at[pl.ds(s * RPT, RPT)],
                        p1_hbm.at[pl.ds(s * RPT, RPT)])


def _mm_body(p0_ref, p1_ref, x_ref, wt_ref, b_ref, o_ref):
    agg = p0_ref[...] + p1_ref[...] + x_ref[...]
    y = jnp.dot(agg, wt_ref[...], preferred_element_type=jnp.float32)
    o_ref[...] = jnp.maximum(y + b_ref[...], 0.0)


_BM = 1000


def _tc_layer(p0, p1, xin, wt, b):
    return pl.pallas_call(
        _mm_body,
        grid=(N_NODES // _BM,),
        in_specs=[
            pl.BlockSpec((_BM, D), lambda i: (i, 0)),
            pl.BlockSpec((_BM, D), lambda i: (i, 0)),
            pl.BlockSpec((_BM, D), lambda i: (i, 0)),
            pl.BlockSpec((D, D), lambda i: (0, 0)),
            pl.BlockSpec((1, D), lambda i: (0, 0)),
        ],
        out_specs=pl.BlockSpec((_BM, D), lambda i: (i, 0)),
        out_shape=jax.ShapeDtypeStruct((N_NODES, D), jnp.float32),
    )(p0, p1, xin, wt, b)


def kernel(x, edge_index, W1, b1, W2, b2):
    src = edge_index[0].astype(jnp.int32)
    dst = edge_index[1].astype(jnp.int32)
    pad = E_PAD - N_EDGES
    src_p = jnp.concatenate([src, jnp.zeros((pad,), jnp.int32)])
    dst_p = jnp.concatenate([dst, jnp.full((pad,), DUMMY, jnp.int32)])
    src_p = src_p.reshape(NC, NS, NCH, CHUNK)
    dst_p = dst_p.reshape(NC, NS, NCH, CHUNK)
    zeros = jnp.zeros((ACC_ROWS, D), jnp.float32)
    wt1 = W1.T
    wt2 = W2.T
    b1r = b1.reshape(1, D)
    b2r = b2.reshape(1, D)

    p0, p1 = _sc_agg(x, src_p, dst_p, zeros)
    h1 = _tc_layer(p0, p1, x, wt1, b1r)
    q0, q1 = _sc_agg(h1, src_p, dst_p, zeros)
    h2 = _tc_layer(q0, q1, h1, wt2, b2r)
    return h2


# speedup vs baseline: 4.6675x; 1.1182x over previous
import functools

import jax
import jax.numpy as jnp
from jax import lax
from jax.experimental import pallas as pl
from jax.experimental.pallas import tpu as pltpu
from jax.experimental.pallas import tpu_sc as plsc

N_NODES = 10000
N_EDGES = 320000
D = 128

NC = 2
NS = 16
NW = NC * NS

CHUNK = 128
NCH = 80
EDGES_PER_W = NCH * CHUNK
E_PAD = EDGES_PER_W * NW
DUMMY = N_NODES
G = 4
NSL = NCH // G
ACC_ROWS = 10112
RPT = ACC_ROWS // NS

_sc_mesh = plsc.VectorSubcoreMesh(core_axis_name="c", subcore_axis_name="s")


@functools.partial(
    pl.kernel,
    mesh=_sc_mesh,
    out_type=[
        jax.ShapeDtypeStruct((ACC_ROWS, D), jnp.float32),
        jax.ShapeDtypeStruct((ACC_ROWS, D), jnp.float32),
    ],
    scratch_types=[
        pltpu.VMEM((2, G, CHUNK), jnp.int32),
        pltpu.VMEM((2, G, CHUNK), jnp.int32),
        pltpu.VMEM((2, CHUNK, D), jnp.float32),
        pltpu.VMEM_SHARED((ACC_ROWS, D), jnp.float32),
        pltpu.SemaphoreType.DMA,
        pltpu.SemaphoreType.DMA,
        pltpu.SemaphoreType.DMA,
        pltpu.SemaphoreType.DMA,
    ],
)
def _sc_agg(h_hbm, src_hbm, dst_hbm, zeros_hbm, p0_hbm, p1_hbm,
            src_sl, dst_sl, rows_v, acc, rs0, rs1, is0, is1):
    c = lax.axis_index("c")
    s = lax.axis_index("s")
    rsem = (rs0, rs1)
    isem = (is0, is1)

    def idx_start(t, p):
        pltpu.make_async_copy(src_hbm.at[c, s, t], src_sl.at[p],
                              isem[p]).start()
        pltpu.make_async_copy(dst_hbm.at[c, s, t], dst_sl.at[p],
                              isem[p]).start()

    def idx_wait(t, p):
        pltpu.make_async_copy(src_hbm.at[c, s, t], src_sl.at[p],
                              isem[p]).wait()
        pltpu.make_async_copy(dst_hbm.at[c, s, t], dst_sl.at[p],
                              isem[p]).wait()

    def row_start(p, g, rb):
        pltpu.make_async_copy(h_hbm.at[src_sl.at[p, g]], rows_v.at[rb],
                              rsem[rb]).start()

    def row_wait(p, g, rb):
        pltpu.make_async_copy(h_hbm.at[src_sl.at[p, g]], rows_v.at[rb],
                              rsem[rb]).wait()

    idx_start(0, 0)
    idx_start(1, 1)
    pltpu.sync_copy(zeros_hbm.at[pl.ds(s * RPT, RPT)],
                    acc.at[pl.ds(s * RPT, RPT)])
    plsc.subcore_barrier()

    idx_wait(0, 0)
    row_start(0, 0, 0)
    row_start(0, 1, 1)

    def outer(k, carry):
        for p in (0, 1):
            t = 2 * k + p
            for g in range(G):
                rb = g % 2
                row_wait(p, g, rb)
                pltpu.sync_copy(rows_v.at[rb], acc.at[dst_sl.at[p, g]],
                                add=True)
                if g < G - 2:
                    row_start(p, g + 2, rb)
                else:
                    @pl.when(t + 1 < NSL)
                    def _(p=p, g=g, rb=rb, t=t):
                        if g == G - 2:
                            idx_wait(t + 1, 1 - p)
                        row_start(1 - p, g + 2 - G, rb)

            @pl.when(t + 2 < NSL)
            def _(p=p, t=t):
                idx_start(t + 2, p)
        return carry

    lax.fori_loop(0, NSL // 2, outer, 0)
    plsc.subcore_barrier()

    @pl.when(c == 0)
    def _():
        pltpu.sync_copy(acc.at[pl.ds(s * RPT, RPT)],
                        p0_hbm.at[pl.ds(s * RPT, RPT)])

    @pl.when(c == 1)
    def _():
        pltpu.sync_copy(acc.at[pl.ds(s * RPT, RPT)],
                        p1_hbm.at[pl.ds(s * RPT, RPT)])


def _mm_body(p0_ref, p1_ref, x_ref, wt_ref, b_ref, o_ref):
    agg = p0_ref[...] + p1_ref[...] + x_ref[...]
    y = jnp.dot(agg, wt_ref[...], preferred_element_type=jnp.float32)
    o_ref[...] = jnp.maximum(y + b_ref[...], 0.0)


_BM = 1000


def _tc_layer(p0, p1, xin, wt, b):
    return pl.pallas_call(
        _mm_body,
        grid=(N_NODES // _BM,),
        in_specs=[
            pl.BlockSpec((_BM, D), lambda i: (i, 0)),
            pl.BlockSpec((_BM, D), lambda i: (i, 0)),
            pl.BlockSpec((_BM, D), lambda i: (i, 0)),
            pl.BlockSpec((D, D), lambda i: (0, 0)),
            pl.BlockSpec((1, D), lambda i: (0, 0)),
        ],
        out_specs=pl.BlockSpec((_BM, D), lambda i: (i, 0)),
        out_shape=jax.ShapeDtypeStruct((N_NODES, D), jnp.float32),
    )(p0, p1, xin, wt, b)


def kernel(x, edge_index, W1, b1, W2, b2):
    src = edge_index[0].astype(jnp.int32)
    dst = edge_index[1].astype(jnp.int32)
    pad = E_PAD - N_EDGES
    src_p = jnp.concatenate([src, jnp.zeros((pad,), jnp.int32)])
    dst_p = jnp.concatenate([dst, jnp.full((pad,), DUMMY, jnp.int32)])
    src_p = src_p.reshape(NC, NS, NSL, G, CHUNK)
    dst_p = dst_p.reshape(NC, NS, NSL, G, CHUNK)
    zeros = jnp.zeros((ACC_ROWS, D), jnp.float32)
    wt1 = W1.T
    wt2 = W2.T
    b1r = b1.reshape(1, D)
    b2r = b2.reshape(1, D)

    p0, p1 = _sc_agg(x, src_p, dst_p, zeros)
    h1 = _tc_layer(p0, p1, x, wt1, b1r)
    q0, q1 = _sc_agg(h1, src_p, dst_p, zeros)
    h2 = _tc_layer(q0, q1, h1, wt2, b2r)
    return h2


# speedup vs baseline: 4.9632x; 1.0634x over previous
import functools

import jax
import jax.numpy as jnp
from jax import lax
from jax.experimental import pallas as pl
from jax.experimental.pallas import tpu as pltpu
from jax.experimental.pallas import tpu_sc as plsc

N_NODES = 10000
N_EDGES = 320000
D = 128

NC = 2
NS = 16
NW = NC * NS

CHUNK = 128
NCH0 = 128
NCH1 = 32
G = 4
NSL0 = NCH0 // G
NSL1 = NCH1 // G
TOT_SLABS = NS * (NSL0 + NSL1)
E_PAD = TOT_SLABS * G * CHUNK
DUMMY = N_NODES
ACC_ROWS = 10112
RPT = ACC_ROWS // NS

_sc_mesh = plsc.VectorSubcoreMesh(core_axis_name="c", subcore_axis_name="s")


@functools.partial(
    pl.kernel,
    mesh=_sc_mesh,
    out_type=[
        jax.ShapeDtypeStruct((ACC_ROWS, D), jnp.float32),
        jax.ShapeDtypeStruct((ACC_ROWS, D), jnp.float32),
    ],
    scratch_types=[
        pltpu.VMEM((2, G, CHUNK), jnp.int32),
        pltpu.VMEM((2, G, CHUNK), jnp.int32),
        pltpu.VMEM((2, CHUNK, D), jnp.float32),
        pltpu.VMEM_SHARED((ACC_ROWS, D), jnp.float32),
        pltpu.SemaphoreType.DMA,
        pltpu.SemaphoreType.DMA,
        pltpu.SemaphoreType.DMA,
        pltpu.SemaphoreType.DMA,
    ],
)
def _sc_agg(h_hbm, src_hbm, dst_hbm, zeros_hbm, p0_hbm, p1_hbm,
            src_sl, dst_sl, rows_v, acc, rs0, rs1, is0, is1):
    c = lax.axis_index("c")
    s = lax.axis_index("s")
    rsem = (rs0, rs1)
    isem = (is0, is1)
    nsl = jnp.where(c == 0, NSL0, NSL1)
    sbase = jnp.where(c == 0, s * NSL0, NS * NSL0 + s * NSL1)

    def idx_start(t, p):
        pltpu.make_async_copy(src_hbm.at[sbase + t], src_sl.at[p],
                              isem[p]).start()
        pltpu.make_async_copy(dst_hbm.at[sbase + t], dst_sl.at[p],
                              isem[p]).start()

    def idx_wait(t, p):
        pltpu.make_async_copy(src_hbm.at[sbase + t], src_sl.at[p],
                              isem[p]).wait()
        pltpu.make_async_copy(dst_hbm.at[sbase + t], dst_sl.at[p],
                              isem[p]).wait()

    def row_start(p, g, rb):
        pltpu.make_async_copy(h_hbm.at[src_sl.at[p, g]], rows_v.at[rb],
                              rsem[rb]).start()

    def row_wait(p, g, rb):
        pltpu.make_async_copy(h_hbm.at[src_sl.at[p, g]], rows_v.at[rb],
                              rsem[rb]).wait()

    idx_start(0, 0)
    idx_start(1, 1)
    pltpu.sync_copy(zeros_hbm.at[pl.ds(s * RPT, RPT)],
                    acc.at[pl.ds(s * RPT, RPT)])
    plsc.subcore_barrier()

    idx_wait(0, 0)
    row_start(0, 0, 0)
    row_start(0, 1, 1)

    def outer(k, carry):
        for p in (0, 1):
            t = 2 * k + p
            for g in range(G):
                rb = g % 2
                row_wait(p, g, rb)
                pltpu.sync_copy(rows_v.at[rb], acc.at[dst_sl.at[p, g]],
                                add=True)
                if g < G - 2:
                    row_start(p, g + 2, rb)
                else:
                    @pl.when(t + 1 < nsl)
                    def _(p=p, g=g, rb=rb, t=t):
                        if g == G - 2:
                            idx_wait(t + 1, 1 - p)
                        row_start(1 - p, g + 2 - G, rb)

            @pl.when(t + 2 < nsl)
            def _(p=p, t=t):
                idx_start(t + 2, p)
        return carry

    lax.fori_loop(0, nsl // 2, outer, 0)
    plsc.subcore_barrier()

    @pl.when(c == 0)
    def _():
        pltpu.sync_copy(acc.at[pl.ds(s * RPT, RPT)],
                        p0_hbm.at[pl.ds(s * RPT, RPT)])

    @pl.when(c == 1)
    def _():
        pltpu.sync_copy(acc.at[pl.ds(s * RPT, RPT)],
                        p1_hbm.at[pl.ds(s * RPT, RPT)])


def _mm_body(p0_ref, p1_ref, x_ref, wt_ref, b_ref, o_ref):
    agg = p0_ref[...] + p1_ref[...] + x_ref[...]
    y = jnp.dot(agg, wt_ref[...], preferred_element_type=jnp.float32)
    o_ref[...] = jnp.maximum(y + b_ref[...], 0.0)


_BM = 1000


def _tc_layer(p0, p1, xin, wt, b):
    return pl.pallas_call(
        _mm_body,
        grid=(N_NODES // _BM,),
        in_specs=[
            pl.BlockSpec((_BM, D), lambda i: (i, 0)),
            pl.BlockSpec((_BM, D), lambda i: (i, 0)),
            pl.BlockSpec((_BM, D), lambda i: (i, 0)),
            pl.BlockSpec((D, D), lambda i: (0, 0)),
            pl.BlockSpec((1, D), lambda i: (0, 0)),
        ],
        out_specs=pl.BlockSpec((_BM, D), lambda i: (i, 0)),
        out_shape=jax.ShapeDtypeStruct((N_NODES, D), jnp.float32),
    )(p0, p1, xin, wt, b)


def kernel(x, edge_index, W1, b1, W2, b2):
    src = edge_index[0].astype(jnp.int32)
    dst = edge_index[1].astype(jnp.int32)
    pad = E_PAD - N_EDGES
    src_p = jnp.concatenate([src, jnp.zeros((pad,), jnp.int32)])
    dst_p = jnp.concatenate([dst, jnp.full((pad,), DUMMY, jnp.int32)])
    src_p = src_p.reshape(TOT_SLABS, G, CHUNK)
    dst_p = dst_p.reshape(TOT_SLABS, G, CHUNK)
    zeros = jnp.zeros((ACC_ROWS, D), jnp.float32)
    wt1 = W1.T
    wt2 = W2.T
    b1r = b1.reshape(1, D)
    b2r = b2.reshape(1, D)

    p0, p1 = _sc_agg(x, src_p, dst_p, zeros)
    h1 = _tc_layer(p0, p1, x, wt1, b1r)
    q0, q1 = _sc_agg(h1, src_p, dst_p, zeros)
    h2 = _tc_layer(q0, q1, h1, wt2, b2r)
    return h2


# speedup vs baseline: 4.9632x; 1.0000x over previous
import functools

import jax
import jax.numpy as jnp
from jax import lax
from jax.experimental import pallas as pl
from jax.experimental.pallas import tpu as pltpu
from jax.experimental.pallas import tpu_sc as plsc

N_NODES = 10000
N_EDGES = 320000
D = 128

NC = 2
NS = 16
NW = NC * NS

CHUNK = 128
NCH0 = 128
NCH1 = 32
G = 4
NSL0 = NCH0 // G
NSL1 = NCH1 // G
TOT_SLABS = NS * (NSL0 + NSL1)
E_PAD = TOT_SLABS * G * CHUNK
DUMMY = N_NODES
ACC_ROWS = 10112
RPT = ACC_ROWS // NS

_sc_mesh = plsc.VectorSubcoreMesh(core_axis_name="c", subcore_axis_name="s")


@functools.partial(
    pl.kernel,
    mesh=_sc_mesh,
    out_type=[
        jax.ShapeDtypeStruct((ACC_ROWS, D), jnp.float32),
        jax.ShapeDtypeStruct((ACC_ROWS, D), jnp.float32),
    ],
    scratch_types=[
        pltpu.VMEM((2, G, CHUNK), jnp.int32),
        pltpu.VMEM((2, G, CHUNK), jnp.int32),
        pltpu.VMEM((2, CHUNK, D), jnp.float32),
        pltpu.VMEM_SHARED((ACC_ROWS, D), jnp.float32),
        pltpu.SemaphoreType.DMA,
        pltpu.SemaphoreType.DMA,
        pltpu.SemaphoreType.DMA,
        pltpu.SemaphoreType.DMA,
    ],
)
def _sc_agg(h_hbm, src_hbm, dst_hbm, zeros_hbm, p0_hbm, p1_hbm,
            src_sl, dst_sl, rows_v, acc, rs0, rs1, is0, is1):
    c = lax.axis_index("c")
    s = lax.axis_index("s")
    rsem = (rs0, rs1)
    isem = (is0, is1)
    nsl = jnp.where(c == 0, NSL0, NSL1)
    sbase = jnp.where(c == 0, s * NSL0, NS * NSL0 + s * NSL1)

    def idx_start(t, p):
        pltpu.make_async_copy(src_hbm.at[sbase + t], src_sl.at[p],
                              isem[p]).start()
        pltpu.make_async_copy(dst_hbm.at[sbase + t], dst_sl.at[p],
                              isem[p]).start()

    def idx_wait(t, p):
        pltpu.make_async_copy(src_hbm.at[sbase + t], src_sl.at[p],
                              isem[p]).wait()
        pltpu.make_async_copy(dst_hbm.at[sbase + t], dst_sl.at[p],
                              isem[p]).wait()

    def row_start(p, g, rb):
        pltpu.make_async_copy(h_hbm.at[src_sl.at[p, g]], rows_v.at[rb],
                              rsem[rb]).start()

    def row_wait(p, g, rb):
        pltpu.make_async_copy(h_hbm.at[src_sl.at[p, g]], rows_v.at[rb],
                              rsem[rb]).wait()

    with jax.named_scope("zero_acc"):
        idx_start(0, 0)
        idx_start(1, 1)
        pltpu.sync_copy(zeros_hbm.at[pl.ds(s * RPT, RPT)],
                        acc.at[pl.ds(s * RPT, RPT)])
        plsc.subcore_barrier()

    idx_wait(0, 0)
    row_start(0, 0, 0)
    row_start(0, 1, 1)

    def outer(k, carry):
        for p in (0, 1):
            t = 2 * k + p
            for g in range(G):
                rb = g % 2
                row_wait(p, g, rb)
                pltpu.sync_copy(rows_v.at[rb], acc.at[dst_sl.at[p, g]],
                                add=True)
                if g < G - 2:
                    row_start(p, g + 2, rb)
                else:
                    @pl.when(t + 1 < nsl)
                    def _(p=p, g=g, rb=rb, t=t):
                        if g == G - 2:
                            idx_wait(t + 1, 1 - p)
                        row_start(1 - p, g + 2 - G, rb)

            @pl.when(t + 2 < nsl)
            def _(p=p, t=t):
                idx_start(t + 2, p)
        return carry

    with jax.named_scope("main_loop"):
        lax.fori_loop(0, nsl // 2, outer, 0)
        plsc.subcore_barrier()

    with jax.named_scope("copy_out"):
        @pl.when(c == 0)
        def _():
            pltpu.sync_copy(acc.at[pl.ds(s * RPT, RPT)],
                            p0_hbm.at[pl.ds(s * RPT, RPT)])

        @pl.when(c == 1)
        def _():
            pltpu.sync_copy(acc.at[pl.ds(s * RPT, RPT)],
                            p1_hbm.at[pl.ds(s * RPT, RPT)])


def _mm_body(p0_ref, p1_ref, x_ref, wt_ref, b_ref, o_ref):
    agg = p0_ref[...] + p1_ref[...] + x_ref[...]
    y = jnp.dot(agg, wt_ref[...], preferred_element_type=jnp.float32)
    o_ref[...] = jnp.maximum(y + b_ref[...], 0.0)


_BM = 1000


def _tc_layer(p0, p1, xin, wt, b):
    return pl.pallas_call(
        _mm_body,
        grid=(N_NODES // _BM,),
        in_specs=[
            pl.BlockSpec((_BM, D), lambda i: (i, 0)),
            pl.BlockSpec((_BM, D), lambda i: (i, 0)),
            pl.BlockSpec((_BM, D), lambda i: (i, 0)),
            pl.BlockSpec((D, D), lambda i: (0, 0)),
            pl.BlockSpec((1, D), lambda i: (0, 0)),
        ],
        out_specs=pl.BlockSpec((_BM, D), lambda i: (i, 0)),
        out_shape=jax.ShapeDtypeStruct((N_NODES, D), jnp.float32),
    )(p0, p1, xin, wt, b)


def kernel(x, edge_index, W1, b1, W2, b2):
    src = edge_index[0].astype(jnp.int32)
    dst = edge_index[1].astype(jnp.int32)
    pad = E_PAD - N_EDGES
    src_p = jnp.concatenate([src, jnp.zeros((pad,), jnp.int32)])
    dst_p = jnp.concatenate([dst, jnp.full((pad,), DUMMY, jnp.int32)])
    src_p = src_p.reshape(TOT_SLABS, G, CHUNK)
    dst_p = dst_p.reshape(TOT_SLABS, G, CHUNK)
    zeros = jnp.zeros((ACC_ROWS, D), jnp.float32)
    wt1 = W1.T
    wt2 = W2.T
    b1r = b1.reshape(1, D)
    b2r = b2.reshape(1, D)

    p0, p1 = _sc_agg(x, src_p, dst_p, zeros)
    h1 = _tc_layer(p0, p1, x, wt1, b1r)
    q0, q1 = _sc_agg(h1, src_p, dst_p, zeros)
    h2 = _tc_layer(q0, q1, h1, wt2, b2r)
    return h2
